# Initial kernel scaffold; baseline (speedup 1.0000x reference)
import jax
import jax.numpy as jnp
from jax.experimental import pallas as pl


def kernel(x, W_enc, W_dec, pre_bias, latent_bias):
    raise NotImplementedError("write your pallas kernel here")


# speedup vs baseline: 1.0461x; 1.0461x over previous
import jax
import jax.numpy as jnp
from jax.experimental import pallas as pl
from jax.experimental.pallas import tpu as pltpu

K_TOP = 64
K_AUX = 256


def _enc_kernel(x_ref, wenc_ref, pb_ref, lb_ref, lat_ref):
    xc = x_ref[...] - pb_ref[...]
    lat_ref[...] = jax.lax.dot_general(
        xc, wenc_ref[...],
        dimension_numbers=(((1,), (1,)), ((), ())),
        preferred_element_type=jnp.float32) + lb_ref[...]


def _dec_kernel(lat_ref, wdec_ref, thr_ref, pb_ref, out_ref, acc_ref):
    jn = pl.program_id(1)
    nj = pl.num_programs(1)

    @pl.when(jn == 0)
    def _init():
        acc_ref[...] = jnp.zeros_like(acc_ref)

    lat = lat_ref[...]
    thr = thr_ref[...]
    lv = jnp.where(lat >= thr, jnp.maximum(lat, 0.0), 0.0).astype(jnp.bfloat16)
    acc_ref[...] += jax.lax.dot_general(
        lv, wdec_ref[...],
        dimension_numbers=(((1,), (1,)), ((), ())),
        preferred_element_type=jnp.float32)

    @pl.when(jn == nj - 1)
    def _finish():
        out_ref[...] = acc_ref[...] + pb_ref[...]


def kernel(x, W_enc, W_dec, pre_bias, latent_bias):
    B, D = x.shape
    N, _ = W_enc.shape
    BM = 256
    NT = 1024
    BM2 = 512
    NT2 = 2048

    pb2 = pre_bias.reshape(1, D)
    lb2 = latent_bias.reshape(1, N)

    lat = pl.pallas_call(
        _enc_kernel,
        grid=(B // BM, N // NT),
        in_specs=[
            pl.BlockSpec((BM, D), lambda i, j: (i, 0)),
            pl.BlockSpec((NT, D), lambda i, j: (j, 0)),
            pl.BlockSpec((1, D), lambda i, j: (0, 0)),
            pl.BlockSpec((1, NT), lambda i, j: (0, j)),
        ],
        out_specs=pl.BlockSpec((BM, NT), lambda i, j: (i, j)),
        out_shape=jax.ShapeDtypeStruct((B, N), jnp.float32),
        compiler_params=pltpu.CompilerParams(
            dimension_semantics=("parallel", "arbitrary")),
    )(x, W_enc, pb2, lb2)

    av, ai = jax.lax.top_k(lat, K_AUX)
    auxv = jnp.maximum(av, 0.0)
    thr = av[:, K_TOP - 1:K_TOP]

    wdec_bf = W_dec.astype(jnp.bfloat16)
    recons = pl.pallas_call(
        _dec_kernel,
        grid=(B // BM2, N // NT2),
        in_specs=[
            pl.BlockSpec((BM2, NT2), lambda i, j: (i, j)),
            pl.BlockSpec((D, NT2), lambda i, j: (0, j)),
            pl.BlockSpec((BM2, 1), lambda i, j: (i, 0)),
            pl.BlockSpec((1, D), lambda i, j: (0, 0)),
        ],
        out_specs=pl.BlockSpec((BM2, D), lambda i, j: (i, 0)),
        out_shape=jax.ShapeDtypeStruct((B, D), jnp.float32),
        scratch_shapes=[pltpu.VMEM((BM2, D), jnp.float32)],
        compiler_params=pltpu.CompilerParams(
            dimension_semantics=("parallel", "arbitrary")),
    )(lat, wdec_bf, thr, pb2)

    return (recons, ai[:, :K_TOP], auxv[:, :K_TOP], ai, auxv)
